# baseline (speedup 1.0000x reference)
import jax
import jax.numpy as jnp
from jax.experimental import pallas as pl
from jax.experimental.pallas import tpu as pltpu

K = 65536
DIM = 128
B = 4096
BR = 2048
NBK = B // BR
NBLK = K // BR
VBR = BR // DIM


def _copy_kernel(s_ref, f_ref, k_ref, v_ref, kv_ref, of_ref, ov_ref):
    i = pl.program_id(0)
    p0 = s_ref[0] // BR
    inside = (i >= p0) & (i < p0 + NBK)

    @pl.when(inside)
    def _():
        of_ref[...] = k_ref[...]
        ov_ref[...] = kv_ref[...]

    @pl.when(~inside)
    def _():
        of_ref[...] = f_ref[...]
        ov_ref[...] = v_ref[...]


def _feat_idx(i, s):
    p0 = s[0] // BR
    inside = (i >= p0) & (i < p0 + NBK)
    dummy = jnp.where(p0 > 0, p0 - 1, NBK)
    return (jnp.where(inside, dummy, i), 0)


def _keys_idx(i, s):
    p0 = s[0] // BR
    inside = (i >= p0) & (i < p0 + NBK)
    return (jnp.where(inside, i - p0, 0), 0)


def _out_idx(i, s):
    return (i, 0)


def kernel(features, vids, keys, key_vids, ptr):
    ptr_arr = jnp.atleast_1d(jnp.asarray(ptr, dtype=jnp.int32))
    vids2d = vids.reshape(K // DIM, DIM)
    kv2d = key_vids.reshape(B // DIM, DIM)

    grid_spec = pltpu.PrefetchScalarGridSpec(
        num_scalar_prefetch=1,
        grid=(NBLK,),
        in_specs=[
            pl.BlockSpec((BR, DIM), _feat_idx),
            pl.BlockSpec((BR, DIM), _keys_idx),
            pl.BlockSpec((VBR, DIM), _feat_idx),
            pl.BlockSpec((VBR, DIM), _keys_idx),
        ],
        out_specs=[
            pl.BlockSpec((BR, DIM), _out_idx),
            pl.BlockSpec((VBR, DIM), _out_idx),
        ],
    )

    features_new, vids_new2d = pl.pallas_call(
        _copy_kernel,
        grid_spec=grid_spec,
        out_shape=[
            jax.ShapeDtypeStruct((K, DIM), features.dtype),
            jax.ShapeDtypeStruct((K // DIM, DIM), vids.dtype),
        ],
        compiler_params=pltpu.CompilerParams(
            dimension_semantics=("parallel",),
        ),
    )(ptr_arr, features, keys, vids2d, kv2d)

    new_ptr = ((ptr_arr[0] + B) % K).astype(jnp.int32)
    return features_new, vids_new2d.reshape(K), new_ptr


# speedup vs baseline: 1.3468x; 1.3468x over previous
import jax
import jax.numpy as jnp
from jax.experimental import pallas as pl
from jax.experimental.pallas import tpu as pltpu

K = 65536
DIM = 128
B = 4096
BR = 8192
NBLK = K // BR
VBR = BR // DIM
VB = B // DIM


def _copy_kernel(s_ref, f_ref, k_ref, v_ref, kv_ref, of_ref, ov_ref):
    i = pl.program_id(0)
    ptr = s_ref[0]
    p0 = ptr // BR
    off = ptr % BR

    of_ref[...] = f_ref[...]
    ov_ref[...] = v_ref[...]

    @pl.when(i == p0)
    def _():
        of_ref[pl.ds(off, B), :] = k_ref[...]
        ov_ref[pl.ds(off // DIM, VB), :] = kv_ref[...]


def kernel(features, vids, keys, key_vids, ptr):
    ptr_arr = jnp.atleast_1d(jnp.asarray(ptr, dtype=jnp.int32))
    vids2d = vids.reshape(K // DIM, DIM)
    kv2d = key_vids.reshape(VB, DIM)

    grid_spec = pltpu.PrefetchScalarGridSpec(
        num_scalar_prefetch=1,
        grid=(NBLK,),
        in_specs=[
            pl.BlockSpec((BR, DIM), lambda i, s: (i, 0)),
            pl.BlockSpec((B, DIM), lambda i, s: (0, 0)),
            pl.BlockSpec((VBR, DIM), lambda i, s: (i, 0)),
            pl.BlockSpec((VB, DIM), lambda i, s: (0, 0)),
        ],
        out_specs=[
            pl.BlockSpec((BR, DIM), lambda i, s: (i, 0)),
            pl.BlockSpec((VBR, DIM), lambda i, s: (i, 0)),
        ],
    )

    features_new, vids_new2d = pl.pallas_call(
        _copy_kernel,
        grid_spec=grid_spec,
        out_shape=[
            jax.ShapeDtypeStruct((K, DIM), features.dtype),
            jax.ShapeDtypeStruct((K // DIM, DIM), vids.dtype),
        ],
    )(ptr_arr, features, keys, vids2d, kv2d)

    new_ptr = ((ptr_arr[0] + B) % K).astype(jnp.int32)
    return features_new, vids_new2d.reshape(K), new_ptr


# speedup vs baseline: 1.3884x; 1.0309x over previous
import jax
import jax.numpy as jnp
from jax.experimental import pallas as pl
from jax.experimental.pallas import tpu as pltpu

K = 65536
DIM = 128
B = 4096
BR = 16384
NBLK = K // BR
VBR = BR // DIM
VB = B // DIM


def _copy_kernel(s_ref, f_ref, k_ref, v_ref, kv_ref, of_ref, ov_ref):
    i = pl.program_id(0)
    ptr = s_ref[0]
    p0 = ptr // BR
    off = ptr % BR

    of_ref[...] = f_ref[...]
    ov_ref[...] = v_ref[...]

    @pl.when(i == p0)
    def _():
        of_ref[pl.ds(off, B), :] = k_ref[...]
        ov_ref[pl.ds(off // DIM, VB), :] = kv_ref[...]


def kernel(features, vids, keys, key_vids, ptr):
    ptr_arr = jnp.atleast_1d(jnp.asarray(ptr, dtype=jnp.int32))
    vids2d = vids.reshape(K // DIM, DIM)
    kv2d = key_vids.reshape(VB, DIM)

    grid_spec = pltpu.PrefetchScalarGridSpec(
        num_scalar_prefetch=1,
        grid=(NBLK,),
        in_specs=[
            pl.BlockSpec((BR, DIM), lambda i, s: (i, 0)),
            pl.BlockSpec((B, DIM), lambda i, s: (0, 0)),
            pl.BlockSpec((VBR, DIM), lambda i, s: (i, 0)),
            pl.BlockSpec((VB, DIM), lambda i, s: (0, 0)),
        ],
        out_specs=[
            pl.BlockSpec((BR, DIM), lambda i, s: (i, 0)),
            pl.BlockSpec((VBR, DIM), lambda i, s: (i, 0)),
        ],
    )

    features_new, vids_new2d = pl.pallas_call(
        _copy_kernel,
        grid_spec=grid_spec,
        out_shape=[
            jax.ShapeDtypeStruct((K, DIM), features.dtype),
            jax.ShapeDtypeStruct((K // DIM, DIM), vids.dtype),
        ],
    )(ptr_arr, features, keys, vids2d, kv2d)

    new_ptr = ((ptr_arr[0] + B) % K).astype(jnp.int32)
    return features_new, vids_new2d.reshape(K), new_ptr
